# baseline (speedup 1.0000x reference)
import jax
import jax.numpy as jnp
from jax.experimental import pallas as pl
from jax.experimental.pallas import tpu as pltpu
from jax.experimental.pallas import tpu_sc as plsc

_VOCAB = 130000
_HIDDEN = 128
_VT = 16384
_GRID = (_VOCAB + _VT - 1) // _VT
_GW = 16


def _sc_gather(ids, embed_w):
    n = ids.shape[0]
    ids2 = ids.reshape(n // _GW, _GW)
    mesh = plsc.VectorSubcoreMesh(core_axis_name="core",
                                  subcore_axis_name="subcore")

    @pl.kernel(out_type=jax.ShapeDtypeStruct((n, _HIDDEN), jnp.float32),
               mesh=mesh)
    def _gather(embed_hbm, ids_hbm, out_hbm):
        def body(i_vmem, o_vmem):
            pltpu.sync_copy(embed_hbm.at[i_vmem.at[0]], o_vmem)

        pltpu.emit_pipeline(
            body,
            grid=(n // _GW,),
            in_specs=[pl.BlockSpec((1, _GW), index_map=lambda i: (i, 0))],
            out_specs=[pl.BlockSpec((_GW, _HIDDEN),
                                    index_map=lambda i: (i, 0))],
            core_axis_name=("core", "subcore"),
            dimension_semantics=(pltpu.PARALLEL,),
        )(ids_hbm, out_hbm)

    return _gather(embed_w, ids2)


def _matmul_body(x_ref, w_ref, b_ref, out_ref):
    acc = jax.lax.dot_general(
        x_ref[...], w_ref[...], (((1,), (1,)), ((), ())),
        preferred_element_type=jnp.float32)
    out_ref[...] = acc + b_ref[...]


def kernel(input_ids, embed_w, head_w, head_b):
    B, Q = input_ids.shape
    n = B * Q
    ids = jnp.clip(input_ids.reshape(n).astype(jnp.int32), 0, _VOCAB - 1)

    x = _sc_gather(ids, embed_w)

    bias2 = head_b.reshape(1, _VOCAB)
    out = pl.pallas_call(
        _matmul_body,
        grid=(_GRID,),
        in_specs=[
            pl.BlockSpec((n, _HIDDEN), lambda j: (0, 0)),
            pl.BlockSpec((_VT, _HIDDEN), lambda j: (j, 0)),
            pl.BlockSpec((1, _VT), lambda j: (0, j)),
        ],
        out_specs=pl.BlockSpec((n, _VT), lambda j: (0, j)),
        out_shape=jax.ShapeDtypeStruct((n, _VOCAB), jnp.float32),
        compiler_params=pltpu.CompilerParams(
            dimension_semantics=(pltpu.PARALLEL,)),
    )(x, head_w, bias2)
    return out.reshape(B, Q, _VOCAB)


# speedup vs baseline: 1.2359x; 1.2359x over previous
import jax
import jax.numpy as jnp
from jax.experimental import pallas as pl
from jax.experimental.pallas import tpu as pltpu

_VOCAB = 130000
_HIDDEN = 128
_VT = 16384
_GRID = (_VOCAB + _VT - 1) // _VT


def _fused_body(ids_ref, embed_ref, w_ref, b_ref, out_ref, x_ref, sem):
    n = x_ref.shape[0]

    @pl.when(pl.program_id(0) == 0)
    def _gather():
        def _start(i, c):
            pltpu.make_async_copy(
                embed_ref.at[pl.ds(ids_ref[i], 1), :],
                x_ref.at[pl.ds(i, 1), :],
                sem).start()
            return c

        jax.lax.fori_loop(0, n, _start, 0)

        def _wait(i, c):
            pltpu.make_async_copy(
                embed_ref.at[pl.ds(ids_ref[i], 1), :],
                x_ref.at[pl.ds(i, 1), :],
                sem).wait()
            return c

        jax.lax.fori_loop(0, n, _wait, 0)

    acc = jax.lax.dot_general(
        x_ref[...], w_ref[...], (((1,), (1,)), ((), ())),
        preferred_element_type=jnp.float32)
    out_ref[...] = acc + b_ref[...]


def kernel(input_ids, embed_w, head_w, head_b):
    B, Q = input_ids.shape
    n = B * Q
    ids = jnp.clip(input_ids.reshape(n).astype(jnp.int32), 0, _VOCAB - 1)

    bias2 = head_b.reshape(1, _VOCAB)
    out = pl.pallas_call(
        _fused_body,
        grid=(_GRID,),
        in_specs=[
            pl.BlockSpec(memory_space=pltpu.SMEM),
            pl.BlockSpec(memory_space=pltpu.MemorySpace.HBM),
            pl.BlockSpec((_VT, _HIDDEN), lambda j: (j, 0)),
            pl.BlockSpec((1, _VT), lambda j: (0, j)),
        ],
        out_specs=pl.BlockSpec((n, _VT), lambda j: (0, j)),
        out_shape=jax.ShapeDtypeStruct((n, _VOCAB), jnp.float32),
        scratch_shapes=[pltpu.VMEM((n, _HIDDEN), jnp.float32),
                        pltpu.SemaphoreType.DMA],
        compiler_params=pltpu.CompilerParams(
            dimension_semantics=(pltpu.ARBITRARY,)),
    )(ids, embed_w, head_w, bias2)
    return out.reshape(B, Q, _VOCAB)
